# baseline (speedup 1.0000x reference)
import functools

import jax
import jax.numpy as jnp
from jax import lax
from jax.experimental import pallas as pl
from jax.experimental.pallas import tpu as pltpu
from jax.experimental.pallas import tpu_sc as plsc

VOCAB = 1000000
B = 4096
L = 200
EMB = 64
HID = 128
NCLS = 2

NC = 2
NS = 16
NW = NC * NS
RPW = B // NW
K1 = 104
K2 = L - K1

NBLK = VOCAB // 128
KPW = NBLK // NW
NREM = NBLK - KPW * NW
VTAIL = VOCAB - NBLK * 128


def _relayout_body(tabt_hbm, tail_hbm, out_hbm, blk, ob, tb, sem_in, sem_out):
    wid = lax.axis_index("c") * NS + lax.axis_index("s")

    def in_copy(c, buf, sem):
        return pltpu.make_async_copy(
            tabt_hbm.at[:, pl.ds(pl.multiple_of(c * 128, 128), 128)],
            blk.at[buf], sem)

    def out_copy(c, buf, sem):
        return pltpu.make_async_copy(
            ob.at[buf],
            out_hbm.at[pl.ds(pl.multiple_of(c * 64, 64), 64)], sem)

    def transpose_block(src, n_pairs, buf):
        def body(j, _):
            for kk in range(4):
                e_idx = jnp.arange(16, dtype=jnp.int32) + 16 * kk
                c0 = jnp.full((16,), 2 * j, jnp.int32)
                c1 = jnp.full((16,), 2 * j + 1, jnp.int32)
                v0 = plsc.load_gather(src, [e_idx, c0])
                v1 = plsc.load_gather(src, [e_idx, c1])
                ob[buf, j, pl.ds(16 * kk, 16)] = v0
                ob[buf, j, pl.ds(64 + 16 * kk, 16)] = v1
            return 0

        lax.fori_loop(0, n_pairs, body, 0)

    in_copy(wid, 0, sem_in).start()

    def outer(k, _):
        c = wid + NW * k
        buf = lax.rem(k, 2)

        @pl.when(k < KPW - 1)
        def _():
            @pl.when(lax.rem(k, 2) == 0)
            def _():
                in_copy(c + NW, 1, sem_in).start()

            @pl.when(lax.rem(k, 2) == 1)
            def _():
                in_copy(c + NW, 0, sem_in).start()

        pltpu.make_async_copy(
            tabt_hbm.at[:, pl.ds(pl.multiple_of(c * 128, 128), 128)],
            blk.at[0], sem_in).wait()

        @pl.when(k >= 2)
        def _():
            out_copy(c - 2 * NW, 0, sem_out).wait()

        @pl.when(lax.rem(k, 2) == 0)
        def _():
            transpose_block(blk.at[0], 64, 0)
            out_copy(c, 0, sem_out).start()

        @pl.when(lax.rem(k, 2) == 1)
        def _():
            transpose_block(blk.at[1], 64, 1)
            out_copy(c, 1, sem_out).start()

        return 0

    lax.fori_loop(0, KPW, outer, 0)
    out_copy(0, 0, sem_out).wait()
    out_copy(0, 1, sem_out).wait()

    @pl.when(wid < NREM)
    def _():
        c = NW * KPW + wid
        pltpu.sync_copy(
            tabt_hbm.at[:, pl.ds(pl.multiple_of(c * 128, 128), 128)],
            blk.at[0])
        transpose_block(blk.at[0], 64, 0)
        pltpu.sync_copy(
            ob.at[0], out_hbm.at[pl.ds(pl.multiple_of(c * 64, 64), 64)])

    @pl.when(wid == NREM)
    def _():
        pltpu.sync_copy(tail_hbm, tb)
        transpose_block(tb, 32, 0)
        pltpu.sync_copy(
            ob.at[0, pl.ds(0, 32)],
            out_hbm.at[pl.ds(NBLK * 64, VTAIL // 2)])


_relayout = functools.partial(
    pl.kernel,
    out_type=jax.ShapeDtypeStruct((VOCAB // 2, 128), jnp.float32),
    mesh=plsc.VectorSubcoreMesh(core_axis_name="c", subcore_axis_name="s"),
    scratch_types=[
        pltpu.VMEM((2, EMB, 128), jnp.float32),
        pltpu.VMEM((2, EMB, 128), jnp.float32),
        pltpu.VMEM((EMB, EMB), jnp.float32),
        pltpu.SemaphoreType.DMA,
        pltpu.SemaphoreType.DMA,
    ],
    compiler_params=pltpu.CompilerParams(
        use_tc_tiling_on_sc=True, needs_layout_passes=False),
)(_relayout_body)


def _sc_pool_body(idx_hbm, table_hbm, out_hbm, idx_v, rows_v, out_v, sem0, sem1):
    wid = lax.axis_index("c") * NS + lax.axis_index("s")
    base = wid * RPW
    pltpu.sync_copy(idx_hbm.at[pl.ds(base * L, RPW * L)], idx_v)

    def fire(r, buf, sem):
        pltpu.make_async_copy(
            table_hbm.at[idx_v.at[pl.ds(r * L, K1)]],
            rows_v.at[buf, pl.ds(0, K1)], sem).start()
        pltpu.make_async_copy(
            table_hbm.at[idx_v.at[pl.ds(r * L + K1, K2)]],
            rows_v.at[buf, pl.ds(K1, K2)], sem).start()

    def drain(r, buf, sem):
        pltpu.make_async_copy(
            table_hbm.at[idx_v.at[pl.ds(r * L, K1)]],
            rows_v.at[buf, pl.ds(0, K1)], sem).wait()
        pltpu.make_async_copy(
            table_hbm.at[idx_v.at[pl.ds(r * L + K1, K2)]],
            rows_v.at[buf, pl.ds(K1, K2)], sem).wait()

    def accum(r, buf):
        def body(l, acc):
            a0, a1, a2, a3 = acc
            a0 = a0 + rows_v[buf, l, pl.ds(0, 16)]
            a1 = a1 + rows_v[buf, l, pl.ds(16, 16)]
            a2 = a2 + rows_v[buf, l, pl.ds(32, 16)]
            a3 = a3 + rows_v[buf, l, pl.ds(48, 16)]
            return (a0, a1, a2, a3)

        z = jnp.zeros((16,), jnp.float32)
        a0, a1, a2, a3 = lax.fori_loop(0, L, body, (z, z, z, z))
        out_v[r, pl.ds(0, 16)] = a0
        out_v[r, pl.ds(16, 16)] = a1
        out_v[r, pl.ds(32, 16)] = a2
        out_v[r, pl.ds(48, 16)] = a3

    fire(0, 0, sem0)

    def outer(i, _):
        r0 = 2 * i
        r1 = r0 + 1
        fire(r1, 1, sem1)
        drain(r0, 0, sem0)
        accum(r0, 0)

        @pl.when(i < RPW // 2 - 1)
        def _():
            fire(r1 + 1, 0, sem0)

        drain(r1, 1, sem1)
        accum(r1, 1)
        return 0

    lax.fori_loop(0, RPW // 2, outer, 0)
    pltpu.sync_copy(out_v, out_hbm.at[pl.ds(base, RPW)])


_sc_pool = functools.partial(
    pl.kernel,
    out_type=jax.ShapeDtypeStruct((B, EMB), jnp.float32),
    mesh=plsc.VectorSubcoreMesh(core_axis_name="c", subcore_axis_name="s"),
    scratch_types=[
        pltpu.VMEM((RPW * L,), jnp.int32),
        pltpu.VMEM((2, L, EMB), jnp.float32),
        pltpu.VMEM((RPW, EMB), jnp.float32),
        pltpu.SemaphoreType.DMA,
        pltpu.SemaphoreType.DMA,
    ],
    compiler_params=pltpu.CompilerParams(use_tc_tiling_on_sc=False),
)(_sc_pool_body)


def _mlp_body(x_ref, len_ref, w1_ref, b1_ref, w2_ref, b2_ref, o_ref):
    x = x_ref[...] / len_ref[...]
    h = jnp.dot(x, w1_ref[...], preferred_element_type=jnp.float32) + b1_ref[...]
    h = jnp.maximum(h, 0.0)
    o_ref[...] = jnp.dot(h, w2_ref[...], preferred_element_type=jnp.float32) + b2_ref[...]


_mlp = pl.pallas_call(
    _mlp_body,
    out_shape=jax.ShapeDtypeStruct((B, HID), jnp.float32),
)


def kernel(batch_inputs, batch_lengths, table, W1, b1, W2, b2):
    lin = _relayout(table.T, table.T[:, NBLK * 128:])
    pooled = _sc_pool(batch_inputs.reshape(B * L), lin.reshape(VOCAB, EMB))
    w2p = jnp.pad(W2, ((0, 0), (0, HID - NCLS)))
    b2p = jnp.pad(b2, (0, HID - NCLS)).reshape(1, HID)
    out = _mlp(pooled, batch_lengths.reshape(B, 1), W1, b1.reshape(1, HID), w2p, b2p)
    return out[:, :NCLS]


# speedup vs baseline: 1.3661x; 1.3661x over previous
import functools

import jax
import jax.numpy as jnp
from jax import lax
from jax.experimental import pallas as pl
from jax.experimental.pallas import tpu as pltpu
from jax.experimental.pallas import tpu_sc as plsc

VOCAB = 1000000
B = 4096
L = 200
EMB = 64
HID = 128
NCLS = 2

NC = 2
NS = 16
NW = NC * NS
RPW = B // NW
K1 = 104
K2 = L - K1

NBLK = VOCAB // 128
KPW = NBLK // NW
NREM = NBLK - KPW * NW
VTAIL = VOCAB - NBLK * 128


def _relayout_body(tabt_hbm, tail_hbm, out_hbm, blk, ob, tb, sem_in, sem_out):
    wid = lax.axis_index("c") * NS + lax.axis_index("s")

    def in_copy(c, buf, sem):
        return pltpu.make_async_copy(
            tabt_hbm.at[:, pl.ds(pl.multiple_of(c * 128, 128), 128)],
            blk.at[buf], sem)

    def out_copy(c, buf, sem):
        return pltpu.make_async_copy(
            ob.at[buf],
            out_hbm.at[pl.ds(pl.multiple_of(c * 64, 64), 64)], sem)

    def transpose_block(src, n_pairs, buf):
        @plsc.parallel_loop(0, n_pairs, 1, unroll=4)
        def body(j):
            for kk in range(4):
                e_idx = jnp.arange(16, dtype=jnp.int32) + 16 * kk
                c0 = jnp.full((16,), 2 * j, jnp.int32)
                c1 = jnp.full((16,), 2 * j + 1, jnp.int32)
                v0 = plsc.load_gather(src, [e_idx, c0])
                v1 = plsc.load_gather(src, [e_idx, c1])
                ob[buf, j, pl.ds(16 * kk, 16)] = v0
                ob[buf, j, pl.ds(64 + 16 * kk, 16)] = v1

    in_copy(wid, 0, sem_in).start()

    def outer(k, _):
        c = wid + NW * k
        buf = lax.rem(k, 2)

        @pl.when(k < KPW - 1)
        def _():
            @pl.when(lax.rem(k, 2) == 0)
            def _():
                in_copy(c + NW, 1, sem_in).start()

            @pl.when(lax.rem(k, 2) == 1)
            def _():
                in_copy(c + NW, 0, sem_in).start()

        pltpu.make_async_copy(
            tabt_hbm.at[:, pl.ds(pl.multiple_of(c * 128, 128), 128)],
            blk.at[0], sem_in).wait()

        @pl.when(k >= 2)
        def _():
            out_copy(c - 2 * NW, 0, sem_out).wait()

        @pl.when(lax.rem(k, 2) == 0)
        def _():
            transpose_block(blk.at[0], 64, 0)
            out_copy(c, 0, sem_out).start()

        @pl.when(lax.rem(k, 2) == 1)
        def _():
            transpose_block(blk.at[1], 64, 1)
            out_copy(c, 1, sem_out).start()

        return 0

    lax.fori_loop(0, KPW, outer, 0)
    out_copy(0, 0, sem_out).wait()
    out_copy(0, 1, sem_out).wait()

    @pl.when(wid < NREM)
    def _():
        c = NW * KPW + wid
        pltpu.sync_copy(
            tabt_hbm.at[:, pl.ds(pl.multiple_of(c * 128, 128), 128)],
            blk.at[0])
        transpose_block(blk.at[0], 64, 0)
        pltpu.sync_copy(
            ob.at[0], out_hbm.at[pl.ds(pl.multiple_of(c * 64, 64), 64)])

    @pl.when(wid == NREM)
    def _():
        pltpu.sync_copy(tail_hbm, tb)
        transpose_block(tb, 32, 0)
        pltpu.sync_copy(
            ob.at[0, pl.ds(0, 32)],
            out_hbm.at[pl.ds(NBLK * 64, VTAIL // 2)])


_relayout = functools.partial(
    pl.kernel,
    out_type=jax.ShapeDtypeStruct((VOCAB // 2, 128), jnp.float32),
    mesh=plsc.VectorSubcoreMesh(core_axis_name="c", subcore_axis_name="s"),
    scratch_types=[
        pltpu.VMEM((2, EMB, 128), jnp.float32),
        pltpu.VMEM((2, EMB, 128), jnp.float32),
        pltpu.VMEM((EMB, EMB), jnp.float32),
        pltpu.SemaphoreType.DMA,
        pltpu.SemaphoreType.DMA,
    ],
    compiler_params=pltpu.CompilerParams(
        use_tc_tiling_on_sc=True, needs_layout_passes=False),
)(_relayout_body)


def _sc_pool_body(idx_hbm, table_hbm, out_hbm, idx_v, rows_v, out_v, sem0, sem1):
    wid = lax.axis_index("c") * NS + lax.axis_index("s")
    base = wid * RPW
    pltpu.sync_copy(idx_hbm.at[pl.ds(base * L, RPW * L)], idx_v)

    def fire(r, buf, sem):
        pltpu.make_async_copy(
            table_hbm.at[idx_v.at[pl.ds(r * L, K1)]],
            rows_v.at[buf, pl.ds(0, K1)], sem).start()
        pltpu.make_async_copy(
            table_hbm.at[idx_v.at[pl.ds(r * L + K1, K2)]],
            rows_v.at[buf, pl.ds(K1, K2)], sem).start()

    def drain(r, buf, sem):
        pltpu.make_async_copy(
            table_hbm.at[idx_v.at[pl.ds(r * L, K1)]],
            rows_v.at[buf, pl.ds(0, K1)], sem).wait()
        pltpu.make_async_copy(
            table_hbm.at[idx_v.at[pl.ds(r * L + K1, K2)]],
            rows_v.at[buf, pl.ds(K1, K2)], sem).wait()

    def accum(r, buf):
        def body(l, acc):
            a0, a1, a2, a3 = acc
            a0 = a0 + rows_v[buf, l, pl.ds(0, 16)]
            a1 = a1 + rows_v[buf, l, pl.ds(16, 16)]
            a2 = a2 + rows_v[buf, l, pl.ds(32, 16)]
            a3 = a3 + rows_v[buf, l, pl.ds(48, 16)]
            return (a0, a1, a2, a3)

        z = jnp.zeros((16,), jnp.float32)
        a0, a1, a2, a3 = lax.fori_loop(0, L, body, (z, z, z, z))
        out_v[r, pl.ds(0, 16)] = a0
        out_v[r, pl.ds(16, 16)] = a1
        out_v[r, pl.ds(32, 16)] = a2
        out_v[r, pl.ds(48, 16)] = a3

    fire(0, 0, sem0)

    def outer(i, _):
        r0 = 2 * i
        r1 = r0 + 1
        fire(r1, 1, sem1)
        drain(r0, 0, sem0)
        accum(r0, 0)

        @pl.when(i < RPW // 2 - 1)
        def _():
            fire(r1 + 1, 0, sem0)

        drain(r1, 1, sem1)
        accum(r1, 1)
        return 0

    lax.fori_loop(0, RPW // 2, outer, 0)
    pltpu.sync_copy(out_v, out_hbm.at[pl.ds(base, RPW)])


_sc_pool = functools.partial(
    pl.kernel,
    out_type=jax.ShapeDtypeStruct((B, EMB), jnp.float32),
    mesh=plsc.VectorSubcoreMesh(core_axis_name="c", subcore_axis_name="s"),
    scratch_types=[
        pltpu.VMEM((RPW * L,), jnp.int32),
        pltpu.VMEM((2, L, EMB), jnp.float32),
        pltpu.VMEM((RPW, EMB), jnp.float32),
        pltpu.SemaphoreType.DMA,
        pltpu.SemaphoreType.DMA,
    ],
    compiler_params=pltpu.CompilerParams(use_tc_tiling_on_sc=False),
)(_sc_pool_body)


def _mlp_body(x_ref, len_ref, w1_ref, b1_ref, w2_ref, b2_ref, o_ref):
    x = x_ref[...] / len_ref[...]
    h = jnp.dot(x, w1_ref[...], preferred_element_type=jnp.float32) + b1_ref[...]
    h = jnp.maximum(h, 0.0)
    o_ref[...] = jnp.dot(h, w2_ref[...], preferred_element_type=jnp.float32) + b2_ref[...]


_mlp = pl.pallas_call(
    _mlp_body,
    out_shape=jax.ShapeDtypeStruct((B, HID), jnp.float32),
)


def kernel(batch_inputs, batch_lengths, table, W1, b1, W2, b2):
    lin = _relayout(table.T, table.T[:, NBLK * 128:])
    pooled = _sc_pool(batch_inputs.reshape(B * L), lin.reshape(VOCAB, EMB))
    w2p = jnp.pad(W2, ((0, 0), (0, HID - NCLS)))
    b2p = jnp.pad(b2, (0, HID - NCLS)).reshape(1, HID)
    out = _mlp(pooled, batch_lengths.reshape(B, 1), W1, b1.reshape(1, HID), w2p, b2p)
    return out[:, :NCLS]


# speedup vs baseline: 1.3717x; 1.0041x over previous
import functools

import jax
import jax.numpy as jnp
from jax import lax
from jax.experimental import pallas as pl
from jax.experimental.pallas import tpu as pltpu
from jax.experimental.pallas import tpu_sc as plsc

VOCAB = 1000000
B = 4096
L = 200
EMB = 64
HID = 128
NCLS = 2

NC = 2
NS = 16
NW = NC * NS
RPW = B // NW
K1 = 104
K2 = L - K1

NBLK = VOCAB // 128
KPW = NBLK // NW
NREM = NBLK - KPW * NW
VTAIL = VOCAB - NBLK * 128


def _relayout_body(tabt_hbm, tail_hbm, out_hbm, blk, ob, tb, sem_in, sem_out):
    wid = lax.axis_index("c") * NS + lax.axis_index("s")

    def in_copy(c, buf, sem):
        return pltpu.make_async_copy(
            tabt_hbm.at[:, pl.ds(pl.multiple_of(c * 128, 128), 128)],
            blk.at[buf], sem)

    def out_copy(c, buf, sem):
        return pltpu.make_async_copy(
            ob.at[buf],
            out_hbm.at[pl.ds(pl.multiple_of(c * 64, 64), 64)], sem)

    def transpose_block(src, n_pairs, buf):
        @plsc.parallel_loop(0, n_pairs, 1, unroll=8)
        def body(j):
            for kk in range(4):
                e_idx = jnp.arange(16, dtype=jnp.int32) + 16 * kk
                c0 = jnp.full((16,), 2 * j, jnp.int32)
                c1 = jnp.full((16,), 2 * j + 1, jnp.int32)
                v0 = plsc.load_gather(src, [e_idx, c0])
                v1 = plsc.load_gather(src, [e_idx, c1])
                ob[buf, j, pl.ds(16 * kk, 16)] = v0
                ob[buf, j, pl.ds(64 + 16 * kk, 16)] = v1

    in_copy(wid, 0, sem_in).start()

    def outer(k, _):
        c = wid + NW * k
        buf = lax.rem(k, 2)

        @pl.when(k < KPW - 1)
        def _():
            @pl.when(lax.rem(k, 2) == 0)
            def _():
                in_copy(c + NW, 1, sem_in).start()

            @pl.when(lax.rem(k, 2) == 1)
            def _():
                in_copy(c + NW, 0, sem_in).start()

        pltpu.make_async_copy(
            tabt_hbm.at[:, pl.ds(pl.multiple_of(c * 128, 128), 128)],
            blk.at[0], sem_in).wait()

        @pl.when(k >= 2)
        def _():
            out_copy(c - 2 * NW, 0, sem_out).wait()

        @pl.when(lax.rem(k, 2) == 0)
        def _():
            transpose_block(blk.at[0], 64, 0)
            out_copy(c, 0, sem_out).start()

        @pl.when(lax.rem(k, 2) == 1)
        def _():
            transpose_block(blk.at[1], 64, 1)
            out_copy(c, 1, sem_out).start()

        return 0

    lax.fori_loop(0, KPW, outer, 0)
    out_copy(0, 0, sem_out).wait()
    out_copy(0, 1, sem_out).wait()

    @pl.when(wid < NREM)
    def _():
        c = NW * KPW + wid
        pltpu.sync_copy(
            tabt_hbm.at[:, pl.ds(pl.multiple_of(c * 128, 128), 128)],
            blk.at[0])
        transpose_block(blk.at[0], 64, 0)
        pltpu.sync_copy(
            ob.at[0], out_hbm.at[pl.ds(pl.multiple_of(c * 64, 64), 64)])

    @pl.when(wid == NREM)
    def _():
        pltpu.sync_copy(tail_hbm, tb)
        transpose_block(tb, 32, 0)
        pltpu.sync_copy(
            ob.at[0, pl.ds(0, 32)],
            out_hbm.at[pl.ds(NBLK * 64, VTAIL // 2)])


_relayout = functools.partial(
    pl.kernel,
    out_type=jax.ShapeDtypeStruct((VOCAB // 2, 128), jnp.float32),
    mesh=plsc.VectorSubcoreMesh(core_axis_name="c", subcore_axis_name="s"),
    scratch_types=[
        pltpu.VMEM((2, EMB, 128), jnp.float32),
        pltpu.VMEM((2, EMB, 128), jnp.float32),
        pltpu.VMEM((EMB, EMB), jnp.float32),
        pltpu.SemaphoreType.DMA,
        pltpu.SemaphoreType.DMA,
    ],
    compiler_params=pltpu.CompilerParams(
        use_tc_tiling_on_sc=True, needs_layout_passes=False),
)(_relayout_body)


def _sc_pool_body(idx_hbm, table_hbm, out_hbm, idx_v, rows_v, out_v, sem0, sem1):
    wid = lax.axis_index("c") * NS + lax.axis_index("s")
    base = wid * RPW
    pltpu.sync_copy(idx_hbm.at[pl.ds(base * L, RPW * L)], idx_v)

    def fire(r, buf, sem):
        pltpu.make_async_copy(
            table_hbm.at[idx_v.at[pl.ds(r * L, K1)]],
            rows_v.at[buf, pl.ds(0, K1)], sem).start()
        pltpu.make_async_copy(
            table_hbm.at[idx_v.at[pl.ds(r * L + K1, K2)]],
            rows_v.at[buf, pl.ds(K1, K2)], sem).start()

    def drain(r, buf, sem):
        pltpu.make_async_copy(
            table_hbm.at[idx_v.at[pl.ds(r * L, K1)]],
            rows_v.at[buf, pl.ds(0, K1)], sem).wait()
        pltpu.make_async_copy(
            table_hbm.at[idx_v.at[pl.ds(r * L + K1, K2)]],
            rows_v.at[buf, pl.ds(K1, K2)], sem).wait()

    def accum(r, buf):
        def body(l, acc):
            a0, a1, a2, a3 = acc
            a0 = a0 + rows_v[buf, l, pl.ds(0, 16)]
            a1 = a1 + rows_v[buf, l, pl.ds(16, 16)]
            a2 = a2 + rows_v[buf, l, pl.ds(32, 16)]
            a3 = a3 + rows_v[buf, l, pl.ds(48, 16)]
            return (a0, a1, a2, a3)

        z = jnp.zeros((16,), jnp.float32)
        a0, a1, a2, a3 = lax.fori_loop(0, L, body, (z, z, z, z))
        out_v[r, pl.ds(0, 16)] = a0
        out_v[r, pl.ds(16, 16)] = a1
        out_v[r, pl.ds(32, 16)] = a2
        out_v[r, pl.ds(48, 16)] = a3

    fire(0, 0, sem0)

    def outer(i, _):
        r0 = 2 * i
        r1 = r0 + 1
        fire(r1, 1, sem1)
        drain(r0, 0, sem0)
        accum(r0, 0)

        @pl.when(i < RPW // 2 - 1)
        def _():
            fire(r1 + 1, 0, sem0)

        drain(r1, 1, sem1)
        accum(r1, 1)
        return 0

    lax.fori_loop(0, RPW // 2, outer, 0)
    pltpu.sync_copy(out_v, out_hbm.at[pl.ds(base, RPW)])


_sc_pool = functools.partial(
    pl.kernel,
    out_type=jax.ShapeDtypeStruct((B, EMB), jnp.float32),
    mesh=plsc.VectorSubcoreMesh(core_axis_name="c", subcore_axis_name="s"),
    scratch_types=[
        pltpu.VMEM((RPW * L,), jnp.int32),
        pltpu.VMEM((2, L, EMB), jnp.float32),
        pltpu.VMEM((RPW, EMB), jnp.float32),
        pltpu.SemaphoreType.DMA,
        pltpu.SemaphoreType.DMA,
    ],
    compiler_params=pltpu.CompilerParams(use_tc_tiling_on_sc=False),
)(_sc_pool_body)


def _mlp_body(x_ref, len_ref, w1_ref, b1_ref, w2_ref, b2_ref, o_ref):
    x = x_ref[...] / len_ref[...]
    h = jnp.dot(x, w1_ref[...], preferred_element_type=jnp.float32) + b1_ref[...]
    h = jnp.maximum(h, 0.0)
    o_ref[...] = jnp.dot(h, w2_ref[...], preferred_element_type=jnp.float32) + b2_ref[...]


_mlp = pl.pallas_call(
    _mlp_body,
    out_shape=jax.ShapeDtypeStruct((B, HID), jnp.float32),
)


def kernel(batch_inputs, batch_lengths, table, W1, b1, W2, b2):
    lin = _relayout(table.T, table.T[:, NBLK * 128:])
    pooled = _sc_pool(batch_inputs.reshape(B * L), lin.reshape(VOCAB, EMB))
    w2p = jnp.pad(W2, ((0, 0), (0, HID - NCLS)))
    b2p = jnp.pad(b2, (0, HID - NCLS)).reshape(1, HID)
    out = _mlp(pooled, batch_lengths.reshape(B, 1), W1, b1.reshape(1, HID), w2p, b2p)
    return out[:, :NCLS]


# speedup vs baseline: 1.7766x; 1.2952x over previous
import functools

import jax
import jax.numpy as jnp
from jax import lax
from jax.experimental import pallas as pl
from jax.experimental.pallas import tpu as pltpu
from jax.experimental.pallas import tpu_sc as plsc

VOCAB = 1000000
B = 4096
L = 200
EMB = 64
HID = 128
NCLS = 2

NC = 2
NS = 16
NW = NC * NS
RPW = B // NW
K1 = 104
K2 = L - K1

NBLK = VOCAB // 128
KPW = NBLK // NW
NREM = NBLK - KPW * NW
VTAIL = VOCAB - NBLK * 128


def _relayout_body(tabt_hbm, tail_hbm, out_hbm, blk, ob, tb, sem_in, sem_out):
    wid = lax.axis_index("c") * NS + lax.axis_index("s")

    def in_copy(c, buf, sem):
        return pltpu.make_async_copy(
            tabt_hbm.at[:, pl.ds(pl.multiple_of(c * 128, 128), 128)],
            blk.at[buf], sem)

    def out_copy(c, buf, sem):
        return pltpu.make_async_copy(
            ob.at[buf],
            out_hbm.at[pl.ds(pl.multiple_of(c * 64, 64), 64)], sem)

    def transpose_block(src, n_pairs, buf):
        lane = jnp.arange(16, dtype=jnp.int32)

        @plsc.parallel_loop(0, n_pairs // 8, 1, unroll=2)
        def body(m):
            cm = jnp.full((16,), 16 * m, jnp.int32)
            rm = jnp.full((16,), 8 * m, jnp.int32)
            for eg in range(4):
                rows_in = eg * 16 + lane
                for d in range(16):
                    s = (lane + d) & 15
                    v = plsc.load_gather(src, [rows_in, cm + s])
                    plsc.store_scatter(
                        ob.at[buf], [rm + (s >> 1), (s & 1) * 64 + rows_in], v)

    in_copy(wid, 0, sem_in).start()

    def outer(k, _):
        c = wid + NW * k
        buf = lax.rem(k, 2)

        @pl.when(k < KPW - 1)
        def _():
            @pl.when(lax.rem(k, 2) == 0)
            def _():
                in_copy(c + NW, 1, sem_in).start()

            @pl.when(lax.rem(k, 2) == 1)
            def _():
                in_copy(c + NW, 0, sem_in).start()

        pltpu.make_async_copy(
            tabt_hbm.at[:, pl.ds(pl.multiple_of(c * 128, 128), 128)],
            blk.at[0], sem_in).wait()

        @pl.when(k >= 2)
        def _():
            out_copy(c - 2 * NW, 0, sem_out).wait()

        @pl.when(lax.rem(k, 2) == 0)
        def _():
            transpose_block(blk.at[0], 64, 0)
            out_copy(c, 0, sem_out).start()

        @pl.when(lax.rem(k, 2) == 1)
        def _():
            transpose_block(blk.at[1], 64, 1)
            out_copy(c, 1, sem_out).start()

        return 0

    lax.fori_loop(0, KPW, outer, 0)
    out_copy(0, 0, sem_out).wait()
    out_copy(0, 1, sem_out).wait()

    @pl.when(wid < NREM)
    def _():
        c = NW * KPW + wid
        pltpu.sync_copy(
            tabt_hbm.at[:, pl.ds(pl.multiple_of(c * 128, 128), 128)],
            blk.at[0])
        transpose_block(blk.at[0], 64, 0)
        pltpu.sync_copy(
            ob.at[0], out_hbm.at[pl.ds(pl.multiple_of(c * 64, 64), 64)])

    @pl.when(wid == NREM)
    def _():
        pltpu.sync_copy(tail_hbm, tb)
        transpose_block(tb, 32, 0)
        pltpu.sync_copy(
            ob.at[0, pl.ds(0, 32)],
            out_hbm.at[pl.ds(NBLK * 64, VTAIL // 2)])


_relayout = functools.partial(
    pl.kernel,
    out_type=jax.ShapeDtypeStruct((VOCAB // 2, 128), jnp.float32),
    mesh=plsc.VectorSubcoreMesh(core_axis_name="c", subcore_axis_name="s"),
    scratch_types=[
        pltpu.VMEM((2, EMB, 128), jnp.float32),
        pltpu.VMEM((2, EMB, 128), jnp.float32),
        pltpu.VMEM((EMB, EMB), jnp.float32),
        pltpu.SemaphoreType.DMA,
        pltpu.SemaphoreType.DMA,
    ],
    compiler_params=pltpu.CompilerParams(
        use_tc_tiling_on_sc=True, needs_layout_passes=False),
)(_relayout_body)


def _sc_pool_body(idx_hbm, table_hbm, out_hbm, idx_v, rows_v, out_v, sem0, sem1):
    wid = lax.axis_index("c") * NS + lax.axis_index("s")
    base = wid * RPW
    pltpu.sync_copy(idx_hbm.at[pl.ds(base * L, RPW * L)], idx_v)

    def fire(r, buf, sem):
        pltpu.make_async_copy(
            table_hbm.at[idx_v.at[pl.ds(r * L, K1)]],
            rows_v.at[buf, pl.ds(0, K1)], sem).start()
        pltpu.make_async_copy(
            table_hbm.at[idx_v.at[pl.ds(r * L + K1, K2)]],
            rows_v.at[buf, pl.ds(K1, K2)], sem).start()

    def drain(r, buf, sem):
        pltpu.make_async_copy(
            table_hbm.at[idx_v.at[pl.ds(r * L, K1)]],
            rows_v.at[buf, pl.ds(0, K1)], sem).wait()
        pltpu.make_async_copy(
            table_hbm.at[idx_v.at[pl.ds(r * L + K1, K2)]],
            rows_v.at[buf, pl.ds(K1, K2)], sem).wait()

    def accum(r, buf):
        def body(l, acc):
            a0, a1, a2, a3 = acc
            a0 = a0 + rows_v[buf, l, pl.ds(0, 16)]
            a1 = a1 + rows_v[buf, l, pl.ds(16, 16)]
            a2 = a2 + rows_v[buf, l, pl.ds(32, 16)]
            a3 = a3 + rows_v[buf, l, pl.ds(48, 16)]
            return (a0, a1, a2, a3)

        z = jnp.zeros((16,), jnp.float32)
        a0, a1, a2, a3 = lax.fori_loop(0, L, body, (z, z, z, z))
        out_v[r, pl.ds(0, 16)] = a0
        out_v[r, pl.ds(16, 16)] = a1
        out_v[r, pl.ds(32, 16)] = a2
        out_v[r, pl.ds(48, 16)] = a3

    fire(0, 0, sem0)

    def outer(i, _):
        r0 = 2 * i
        r1 = r0 + 1
        fire(r1, 1, sem1)
        drain(r0, 0, sem0)
        accum(r0, 0)

        @pl.when(i < RPW // 2 - 1)
        def _():
            fire(r1 + 1, 0, sem0)

        drain(r1, 1, sem1)
        accum(r1, 1)
        return 0

    lax.fori_loop(0, RPW // 2, outer, 0)
    pltpu.sync_copy(out_v, out_hbm.at[pl.ds(base, RPW)])


_sc_pool = functools.partial(
    pl.kernel,
    out_type=jax.ShapeDtypeStruct((B, EMB), jnp.float32),
    mesh=plsc.VectorSubcoreMesh(core_axis_name="c", subcore_axis_name="s"),
    scratch_types=[
        pltpu.VMEM((RPW * L,), jnp.int32),
        pltpu.VMEM((2, L, EMB), jnp.float32),
        pltpu.VMEM((RPW, EMB), jnp.float32),
        pltpu.SemaphoreType.DMA,
        pltpu.SemaphoreType.DMA,
    ],
    compiler_params=pltpu.CompilerParams(use_tc_tiling_on_sc=False),
)(_sc_pool_body)


def _mlp_body(x_ref, len_ref, w1_ref, b1_ref, w2_ref, b2_ref, o_ref):
    x = x_ref[...] / len_ref[...]
    h = jnp.dot(x, w1_ref[...], preferred_element_type=jnp.float32) + b1_ref[...]
    h = jnp.maximum(h, 0.0)
    o_ref[...] = jnp.dot(h, w2_ref[...], preferred_element_type=jnp.float32) + b2_ref[...]


_mlp = pl.pallas_call(
    _mlp_body,
    out_shape=jax.ShapeDtypeStruct((B, HID), jnp.float32),
)


def kernel(batch_inputs, batch_lengths, table, W1, b1, W2, b2):
    lin = _relayout(table.T, table.T[:, NBLK * 128:])
    pooled = _sc_pool(batch_inputs.reshape(B * L), lin.reshape(VOCAB, EMB))
    w2p = jnp.pad(W2, ((0, 0), (0, HID - NCLS)))
    b2p = jnp.pad(b2, (0, HID - NCLS)).reshape(1, HID)
    out = _mlp(pooled, batch_lengths.reshape(B, 1), W1, b1.reshape(1, HID), w2p, b2p)
    return out[:, :NCLS]


# speedup vs baseline: 3.1488x; 1.7723x over previous
import functools

import jax
import jax.numpy as jnp
from jax import lax
from jax.experimental import pallas as pl
from jax.experimental.pallas import tpu as pltpu
from jax.experimental.pallas import tpu_sc as plsc

VOCAB = 1000000
B = 4096
L = 200
EMB = 64
HID = 128
NCLS = 2

NC = 2
NS = 16
NW = NC * NS
RPW = B // NW
K1 = 104
K2 = L - K1

NBLK = VOCAB // 128
KPW = NBLK // NW
NREM = NBLK - KPW * NW
VTAIL = VOCAB - NBLK * 128


def _relayout_body(tabt_hbm, tail_hbm, out_hbm, blk, ob, tb, sem_in, sem_out):
    wid = lax.axis_index("c") * NS + lax.axis_index("s")

    def in_copy(c, buf, sem):
        return pltpu.make_async_copy(
            tabt_hbm.at[:, pl.ds(pl.multiple_of(c * 128, 128), 128)],
            blk.at[buf], sem)

    def out_copy(c, buf, sem):
        return pltpu.make_async_copy(
            ob.at[buf],
            out_hbm.at[pl.ds(pl.multiple_of(c * 64, 64), 64)], sem)

    def transpose_block(src, n_pairs, buf):
        lane = jnp.arange(16, dtype=jnp.int32)

        @plsc.parallel_loop(0, n_pairs // 8, 1, unroll=4)
        def body(m):
            cm = jnp.full((16,), 16 * m, jnp.int32)
            rm = jnp.full((16,), 8 * m, jnp.int32)
            for eg in range(4):
                rows_in = eg * 16 + lane
                for d in range(16):
                    s = (lane + d) & 15
                    v = plsc.load_gather(src, [rows_in, cm + s])
                    plsc.store_scatter(
                        ob.at[buf], [rm + (s >> 1), (s & 1) * 64 + rows_in], v)

    in_copy(wid, 0, sem_in).start()

    def outer(k, _):
        c = wid + NW * k
        buf = lax.rem(k, 2)

        @pl.when(k < KPW - 1)
        def _():
            @pl.when(lax.rem(k, 2) == 0)
            def _():
                in_copy(c + NW, 1, sem_in).start()

            @pl.when(lax.rem(k, 2) == 1)
            def _():
                in_copy(c + NW, 0, sem_in).start()

        pltpu.make_async_copy(
            tabt_hbm.at[:, pl.ds(pl.multiple_of(c * 128, 128), 128)],
            blk.at[0], sem_in).wait()

        @pl.when(k >= 2)
        def _():
            out_copy(c - 2 * NW, 0, sem_out).wait()

        @pl.when(lax.rem(k, 2) == 0)
        def _():
            transpose_block(blk.at[0], 64, 0)
            out_copy(c, 0, sem_out).start()

        @pl.when(lax.rem(k, 2) == 1)
        def _():
            transpose_block(blk.at[1], 64, 1)
            out_copy(c, 1, sem_out).start()

        return 0

    lax.fori_loop(0, KPW, outer, 0)
    out_copy(0, 0, sem_out).wait()
    out_copy(0, 1, sem_out).wait()

    @pl.when(wid < NREM)
    def _():
        c = NW * KPW + wid
        pltpu.sync_copy(
            tabt_hbm.at[:, pl.ds(pl.multiple_of(c * 128, 128), 128)],
            blk.at[0])
        transpose_block(blk.at[0], 64, 0)
        pltpu.sync_copy(
            ob.at[0], out_hbm.at[pl.ds(pl.multiple_of(c * 64, 64), 64)])

    @pl.when(wid == NREM)
    def _():
        pltpu.sync_copy(tail_hbm, tb)
        transpose_block(tb, 32, 0)
        pltpu.sync_copy(
            ob.at[0, pl.ds(0, 32)],
            out_hbm.at[pl.ds(NBLK * 64, VTAIL // 2)])


_relayout = functools.partial(
    pl.kernel,
    out_type=jax.ShapeDtypeStruct((VOCAB // 2, 128), jnp.float32),
    mesh=plsc.VectorSubcoreMesh(core_axis_name="c", subcore_axis_name="s"),
    scratch_types=[
        pltpu.VMEM((2, EMB, 128), jnp.float32),
        pltpu.VMEM((2, EMB, 128), jnp.float32),
        pltpu.VMEM((EMB, EMB), jnp.float32),
        pltpu.SemaphoreType.DMA,
        pltpu.SemaphoreType.DMA,
    ],
    compiler_params=pltpu.CompilerParams(
        use_tc_tiling_on_sc=True, needs_layout_passes=False),
)(_relayout_body)


def _sc_pool_body(idx_hbm, table_hbm, out_hbm, idx_v, rows_v, out_v, sem0, sem1):
    wid = lax.axis_index("c") * NS + lax.axis_index("s")
    base = wid * RPW
    pltpu.sync_copy(idx_hbm.at[pl.ds(base * L, RPW * L)], idx_v)

    def fire(r, buf, sem):
        pltpu.make_async_copy(
            table_hbm.at[idx_v.at[pl.ds(r * L, K1)]],
            rows_v.at[buf, pl.ds(0, K1)], sem).start()
        pltpu.make_async_copy(
            table_hbm.at[idx_v.at[pl.ds(r * L + K1, K2)]],
            rows_v.at[buf, pl.ds(K1, K2)], sem).start()

    def drain(r, buf, sem):
        pltpu.make_async_copy(
            table_hbm.at[idx_v.at[pl.ds(r * L, K1)]],
            rows_v.at[buf, pl.ds(0, K1)], sem).wait()
        pltpu.make_async_copy(
            table_hbm.at[idx_v.at[pl.ds(r * L + K1, K2)]],
            rows_v.at[buf, pl.ds(K1, K2)], sem).wait()

    def accum(r, buf):
        def body(l, acc):
            a0, a1, a2, a3 = acc
            a0 = a0 + rows_v[buf, l, pl.ds(0, 16)]
            a1 = a1 + rows_v[buf, l, pl.ds(16, 16)]
            a2 = a2 + rows_v[buf, l, pl.ds(32, 16)]
            a3 = a3 + rows_v[buf, l, pl.ds(48, 16)]
            return (a0, a1, a2, a3)

        z = jnp.zeros((16,), jnp.float32)
        a0, a1, a2, a3 = lax.fori_loop(0, L, body, (z, z, z, z))
        out_v[r, pl.ds(0, 16)] = a0
        out_v[r, pl.ds(16, 16)] = a1
        out_v[r, pl.ds(32, 16)] = a2
        out_v[r, pl.ds(48, 16)] = a3

    fire(0, 0, sem0)

    def outer(i, _):
        r0 = 2 * i
        r1 = r0 + 1
        fire(r1, 1, sem1)
        drain(r0, 0, sem0)
        accum(r0, 0)

        @pl.when(i < RPW // 2 - 1)
        def _():
            fire(r1 + 1, 0, sem0)

        drain(r1, 1, sem1)
        accum(r1, 1)
        return 0

    lax.fori_loop(0, RPW // 2, outer, 0)
    pltpu.sync_copy(out_v, out_hbm.at[pl.ds(base, RPW)])


_sc_pool = functools.partial(
    pl.kernel,
    out_type=jax.ShapeDtypeStruct((B, EMB), jnp.float32),
    mesh=plsc.VectorSubcoreMesh(core_axis_name="c", subcore_axis_name="s"),
    scratch_types=[
        pltpu.VMEM((RPW * L,), jnp.int32),
        pltpu.VMEM((2, L, EMB), jnp.float32),
        pltpu.VMEM((RPW, EMB), jnp.float32),
        pltpu.SemaphoreType.DMA,
        pltpu.SemaphoreType.DMA,
    ],
    compiler_params=pltpu.CompilerParams(use_tc_tiling_on_sc=False),
)(_sc_pool_body)


def _mlp_body(x_ref, len_ref, w1_ref, b1_ref, w2_ref, b2_ref, o_ref):
    x = x_ref[...] / len_ref[...]
    h = jnp.dot(x, w1_ref[...], preferred_element_type=jnp.float32) + b1_ref[...]
    h = jnp.maximum(h, 0.0)
    o_ref[...] = jnp.dot(h, w2_ref[...], preferred_element_type=jnp.float32) + b2_ref[...]


_mlp = pl.pallas_call(
    _mlp_body,
    out_shape=jax.ShapeDtypeStruct((B, HID), jnp.float32),
)


def kernel(batch_inputs, batch_lengths, table, W1, b1, W2, b2):
    lin = _relayout(table.T, table.T[:, NBLK * 128:])
    pooled = _sc_pool(batch_inputs.reshape(B * L), lin.reshape(VOCAB, EMB))
    w2p = jnp.pad(W2, ((0, 0), (0, HID - NCLS)))
    b2p = jnp.pad(b2, (0, HID - NCLS)).reshape(1, HID)
    out = _mlp(pooled, batch_lengths.reshape(B, 1), W1, b1.reshape(1, HID), w2p, b2p)
    return out[:, :NCLS]


# speedup vs baseline: 3.1581x; 1.0030x over previous
import functools

import jax
import jax.numpy as jnp
from jax import lax
from jax.experimental import pallas as pl
from jax.experimental.pallas import tpu as pltpu
from jax.experimental.pallas import tpu_sc as plsc

VOCAB = 1000000
B = 4096
L = 200
EMB = 64
HID = 128
NCLS = 2

NC = 2
NS = 16
NW = NC * NS
RPW = B // NW
K1 = 104
K2 = L - K1

NBLK = VOCAB // 128
KPW = NBLK // NW
NREM = NBLK - KPW * NW
VTAIL = VOCAB - NBLK * 128


def _relayout_body(tabt_hbm, tail_hbm, out_hbm, blk, ob, tb,
                   sem_in0, sem_in1, sem_out0, sem_out1):
    wid = lax.axis_index("c") * NS + lax.axis_index("s")

    def in_copy(c, buf, sem):
        return pltpu.make_async_copy(
            tabt_hbm.at[:, pl.ds(pl.multiple_of(c * 128, 128), 128)],
            blk.at[buf], sem)

    def out_copy(c, buf, sem):
        return pltpu.make_async_copy(
            ob.at[buf],
            out_hbm.at[pl.ds(pl.multiple_of(c * 64, 64), 64)], sem)

    def transpose_block(src, n_pairs, buf):
        lane = jnp.arange(16, dtype=jnp.int32)

        @plsc.parallel_loop(0, n_pairs // 8, 1, unroll=4)
        def body(m):
            cm = jnp.full((16,), 16 * m, jnp.int32)
            rm = jnp.full((16,), 8 * m, jnp.int32)
            for eg in range(4):
                rows_in = eg * 16 + lane
                for d in range(16):
                    s = (lane + d) & 15
                    v = plsc.load_gather(src, [rows_in, cm + s])
                    plsc.store_scatter(
                        ob.at[buf], [rm + (s >> 1), (s & 1) * 64 + rows_in], v)

    in_copy(wid, 0, sem_in0).start()

    def outer(k, _):
        c = wid + NW * k

        @pl.when(lax.rem(k, 2) == 0)
        def _():
            @pl.when(k < KPW - 1)
            def _():
                in_copy(c + NW, 1, sem_in1).start()

            in_copy(c, 0, sem_in0).wait()

            @pl.when(k >= 2)
            def _():
                out_copy(c - 2 * NW, 0, sem_out0).wait()

            transpose_block(blk.at[0], 64, 0)
            out_copy(c, 0, sem_out0).start()

        @pl.when(lax.rem(k, 2) == 1)
        def _():
            @pl.when(k < KPW - 1)
            def _():
                in_copy(c + NW, 0, sem_in0).start()

            in_copy(c, 1, sem_in1).wait()

            @pl.when(k >= 2)
            def _():
                out_copy(c - 2 * NW, 1, sem_out1).wait()

            transpose_block(blk.at[1], 64, 1)
            out_copy(c, 1, sem_out1).start()

        return 0

    lax.fori_loop(0, KPW, outer, 0)
    out_copy(0, 0, sem_out0).wait()
    out_copy(0, 1, sem_out1).wait()

    @pl.when(wid < NREM)
    def _():
        c = NW * KPW + wid
        pltpu.sync_copy(
            tabt_hbm.at[:, pl.ds(pl.multiple_of(c * 128, 128), 128)],
            blk.at[0])
        transpose_block(blk.at[0], 64, 0)
        pltpu.sync_copy(
            ob.at[0], out_hbm.at[pl.ds(pl.multiple_of(c * 64, 64), 64)])

    @pl.when(wid == NREM)
    def _():
        pltpu.sync_copy(tail_hbm, tb)
        transpose_block(tb, 32, 0)
        pltpu.sync_copy(
            ob.at[0, pl.ds(0, 32)],
            out_hbm.at[pl.ds(NBLK * 64, VTAIL // 2)])


_relayout = functools.partial(
    pl.kernel,
    out_type=jax.ShapeDtypeStruct((VOCAB // 2, 128), jnp.float32),
    mesh=plsc.VectorSubcoreMesh(core_axis_name="c", subcore_axis_name="s"),
    scratch_types=[
        pltpu.VMEM((2, EMB, 128), jnp.float32),
        pltpu.VMEM((2, EMB, 128), jnp.float32),
        pltpu.VMEM((EMB, EMB), jnp.float32),
        pltpu.SemaphoreType.DMA,
        pltpu.SemaphoreType.DMA,
        pltpu.SemaphoreType.DMA,
        pltpu.SemaphoreType.DMA,
    ],
    compiler_params=pltpu.CompilerParams(
        use_tc_tiling_on_sc=True, needs_layout_passes=False),
)(_relayout_body)


def _sc_pool_body(idx_hbm, table_hbm, out_hbm, idx_v, rows_v, out_v, sem0, sem1):
    wid = lax.axis_index("c") * NS + lax.axis_index("s")
    base = wid * RPW
    pltpu.sync_copy(idx_hbm.at[pl.ds(base * L, RPW * L)], idx_v)

    def fire(r, buf, sem):
        pltpu.make_async_copy(
            table_hbm.at[idx_v.at[pl.ds(r * L, K1)]],
            rows_v.at[buf, pl.ds(0, K1)], sem).start()
        pltpu.make_async_copy(
            table_hbm.at[idx_v.at[pl.ds(r * L + K1, K2)]],
            rows_v.at[buf, pl.ds(K1, K2)], sem).start()

    def drain(r, buf, sem):
        pltpu.make_async_copy(
            table_hbm.at[idx_v.at[pl.ds(r * L, K1)]],
            rows_v.at[buf, pl.ds(0, K1)], sem).wait()
        pltpu.make_async_copy(
            table_hbm.at[idx_v.at[pl.ds(r * L + K1, K2)]],
            rows_v.at[buf, pl.ds(K1, K2)], sem).wait()

    def accum(r, buf):
        def body(l, acc):
            a0, a1, a2, a3 = acc
            a0 = a0 + rows_v[buf, l, pl.ds(0, 16)]
            a1 = a1 + rows_v[buf, l, pl.ds(16, 16)]
            a2 = a2 + rows_v[buf, l, pl.ds(32, 16)]
            a3 = a3 + rows_v[buf, l, pl.ds(48, 16)]
            return (a0, a1, a2, a3)

        z = jnp.zeros((16,), jnp.float32)
        a0, a1, a2, a3 = lax.fori_loop(0, L, body, (z, z, z, z))
        out_v[r, pl.ds(0, 16)] = a0
        out_v[r, pl.ds(16, 16)] = a1
        out_v[r, pl.ds(32, 16)] = a2
        out_v[r, pl.ds(48, 16)] = a3

    fire(0, 0, sem0)

    def outer(i, _):
        r0 = 2 * i
        r1 = r0 + 1
        fire(r1, 1, sem1)
        drain(r0, 0, sem0)
        accum(r0, 0)

        @pl.when(i < RPW // 2 - 1)
        def _():
            fire(r1 + 1, 0, sem0)

        drain(r1, 1, sem1)
        accum(r1, 1)
        return 0

    lax.fori_loop(0, RPW // 2, outer, 0)
    pltpu.sync_copy(out_v, out_hbm.at[pl.ds(base, RPW)])


_sc_pool = functools.partial(
    pl.kernel,
    out_type=jax.ShapeDtypeStruct((B, EMB), jnp.float32),
    mesh=plsc.VectorSubcoreMesh(core_axis_name="c", subcore_axis_name="s"),
    scratch_types=[
        pltpu.VMEM((RPW * L,), jnp.int32),
        pltpu.VMEM((2, L, EMB), jnp.float32),
        pltpu.VMEM((RPW, EMB), jnp.float32),
        pltpu.SemaphoreType.DMA,
        pltpu.SemaphoreType.DMA,
    ],
    compiler_params=pltpu.CompilerParams(use_tc_tiling_on_sc=False),
)(_sc_pool_body)


def _mlp_body(x_ref, len_ref, w1_ref, b1_ref, w2_ref, b2_ref, o_ref):
    x = x_ref[...] / len_ref[...]
    h = jnp.dot(x, w1_ref[...], preferred_element_type=jnp.float32) + b1_ref[...]
    h = jnp.maximum(h, 0.0)
    o_ref[...] = jnp.dot(h, w2_ref[...], preferred_element_type=jnp.float32) + b2_ref[...]


_mlp = pl.pallas_call(
    _mlp_body,
    out_shape=jax.ShapeDtypeStruct((B, HID), jnp.float32),
)


def kernel(batch_inputs, batch_lengths, table, W1, b1, W2, b2):
    lin = _relayout(table.T, table.T[:, NBLK * 128:])
    pooled = _sc_pool(batch_inputs.reshape(B * L), lin.reshape(VOCAB, EMB))
    w2p = jnp.pad(W2, ((0, 0), (0, HID - NCLS)))
    b2p = jnp.pad(b2, (0, HID - NCLS)).reshape(1, HID)
    out = _mlp(pooled, batch_lengths.reshape(B, 1), W1, b1.reshape(1, HID), w2p, b2p)
    return out[:, :NCLS]
